# baseline (speedup 1.0000x reference)
import functools

import jax
import jax.numpy as jnp
from jax import lax
from jax.experimental import pallas as pl
from jax.experimental.pallas import tpu as pltpu
from jax.experimental.pallas import tpu_sc as plsc

N = 2048
D = 2048
H = 2048
OUT = 2048
E = 8
K = 2

BM = 512
BM_BITS = 9
M_PAD = N * K + E * BM
NB = M_PAD // BM
NBP = 16

TB = 512
NTB = N // TB

NC = 2
NS = 16
NW = NC * NS
TW = N // NW
LANES = 16


def _routing_body(x_ref, gw_ref, gb_ref, e1_ref, e2_ref, r1_ref, r2_ref,
                  w1_ref, w2_ref, cnt_ref, carry_ref):
    i = pl.program_id(0)

    @pl.when(i == 0)
    def _():
        carry_ref[...] = jnp.zeros_like(carry_ref)

    s = jnp.dot(x_ref[...], gw_ref[...], preferred_element_type=jnp.float32)
    s = s + gb_ref[...]
    idx = lax.broadcasted_iota(jnp.int32, (TB, E), 1)

    m1 = jnp.max(s, axis=1, keepdims=True)
    i1 = jnp.min(jnp.where(s == m1, idx, E), axis=1, keepdims=True)
    s2 = jnp.where(idx == i1, -jnp.inf, s)
    m2 = jnp.max(s2, axis=1, keepdims=True)
    i2 = jnp.min(jnp.where(s2 == m2, idx, E), axis=1, keepdims=True)

    p = jnp.exp(s - m1)
    z = jnp.sum(p, axis=1, keepdims=True)
    e2v = jnp.exp(m2 - m1)
    denom = 1.0 + e2v + 1e-8 * z
    w1_ref[...] = 1.0 / denom
    w2_ref[...] = e2v / denom

    oh1 = (idx == i1).astype(jnp.float32)
    oh2 = (idx == i2).astype(jnp.float32)
    maskf = oh1 + oh2

    row = lax.broadcasted_iota(jnp.int32, (TB, TB), 0)
    col = lax.broadcasted_iota(jnp.int32, (TB, TB), 1)
    tri = (col < row).astype(jnp.float32)
    local = jnp.dot(tri, maskf, preferred_element_type=jnp.float32)
    rank = local + carry_ref[...]
    r1_ref[...] = jnp.sum(oh1 * rank, axis=1, keepdims=True).astype(jnp.int32)
    r2_ref[...] = jnp.sum(oh2 * rank, axis=1, keepdims=True).astype(jnp.int32)
    e1_ref[...] = i1
    e2_ref[...] = i2

    carry_new = carry_ref[...] + jnp.sum(maskf, axis=0, keepdims=True)
    carry_ref[...] = carry_new

    @pl.when(i == NTB - 1)
    def _():
        cnt_ref[...] = carry_new.astype(jnp.int32)


def _routing(x, gate_W, gate_b):
    col = pl.BlockSpec((TB, 1), lambda i: (i, 0))
    return pl.pallas_call(
        _routing_body,
        grid=(NTB,),
        in_specs=[
            pl.BlockSpec((TB, D), lambda i: (i, 0)),
            pl.BlockSpec((D, E), lambda i: (0, 0)),
            pl.BlockSpec((1, E), lambda i: (0, 0)),
        ],
        out_specs=[col, col, col, col, col, col,
                   pl.BlockSpec((1, E), lambda i: (0, 0))],
        out_shape=[
            jax.ShapeDtypeStruct((N, 1), jnp.int32),
            jax.ShapeDtypeStruct((N, 1), jnp.int32),
            jax.ShapeDtypeStruct((N, 1), jnp.int32),
            jax.ShapeDtypeStruct((N, 1), jnp.int32),
            jax.ShapeDtypeStruct((N, 1), jnp.float32),
            jax.ShapeDtypeStruct((N, 1), jnp.float32),
            jax.ShapeDtypeStruct((1, E), jnp.int32),
        ],
        scratch_shapes=[pltpu.VMEM((1, E), jnp.float32)],
        compiler_params=pltpu.CompilerParams(
            dimension_semantics=("arbitrary",)),
    )(x, gate_W, gate_b)


@functools.cache
def _build_dispatch():
    mesh = plsc.VectorSubcoreMesh(core_axis_name="c", subcore_axis_name="s")

    @functools.partial(
        pl.kernel,
        mesh=mesh,
        out_type=[
            jax.ShapeDtypeStruct((M_PAD, D), jnp.float32),
            jax.ShapeDtypeStruct((N,), jnp.int32),
            jax.ShapeDtypeStruct((N,), jnp.int32),
            jax.ShapeDtypeStruct((NBP,), jnp.int32),
            jax.ShapeDtypeStruct((8,), jnp.int32),
        ],
        scratch_types=[
            pltpu.VMEM((LANES,), jnp.int32),
            pltpu.VMEM((LANES,), jnp.int32),
            pltpu.VMEM((NBP,), jnp.int32),
            pltpu.VMEM((TW,), jnp.int32),
            pltpu.VMEM((TW,), jnp.int32),
            pltpu.VMEM((TW,), jnp.int32),
            pltpu.VMEM((TW,), jnp.int32),
            pltpu.VMEM((TW,), jnp.int32),
            pltpu.VMEM((TW,), jnp.int32),
            pltpu.VMEM((TW // LANES, LANES), jnp.int32),
            pltpu.VMEM((TW // LANES, LANES), jnp.int32),
            pltpu.VMEM((3, LANES, D), jnp.float32),
            pltpu.SemaphoreType.DMA,
            pltpu.SemaphoreType.DMA,
            pltpu.SemaphoreType.DMA,
            pltpu.SemaphoreType.DMA,
        ],
        compiler_params=pltpu.CompilerParams(needs_layout_passes=False),
    )
    def dispatch(e1_hbm, e2_hbm, r1_hbm, r2_hbm, cnt_hbm,
                 x_hbm, xs_hbm, rows1_hbm, rows2_hbm, be_hbm, nab_hbm,
                 cnt_v, off_v, be_v, e1_v, e2_v, r1_v, r2_v,
                 rows1_v, rows2_v, rows1_m, rows2_m, xbuf,
                 sem1, sem2, sem3, sem4):
        wid = lax.axis_index("s") * NC + lax.axis_index("c")
        tok0 = wid * TW

        sems = [sem1, sem2, sem3]
        rd = [pltpu.async_copy(
            x_hbm.at[pl.ds(tok0 + b * LANES, LANES)], xbuf.at[b], sems[b])
            for b in range(3)]

        hc = (pltpu.async_copy(cnt_hbm, cnt_v.at[pl.ds(0, E)], sem4),
              pltpu.async_copy(e1_hbm.at[pl.ds(tok0, TW)], e1_v, sem4),
              pltpu.async_copy(e2_hbm.at[pl.ds(tok0, TW)], e2_v, sem4),
              pltpu.async_copy(r1_hbm.at[pl.ds(tok0, TW)], r1_v, sem4),
              pltpu.async_copy(r2_hbm.at[pl.ds(tok0, TW)], r2_v, sem4))
        for h in hc:
            h.wait()
        lane = lax.iota(jnp.int32, LANES)
        c = cnt_v[...]
        cpad = jnp.where(lane < E, ((c + (BM - 1)) >> BM_BITS) << BM_BITS, 0)
        cs = plsc.cumsum(cpad)
        off = cs - cpad
        off_v[...] = off

        @pl.when(wid == 0)
        def _():
            for cc in range(NBP // LANES):
                bs = (lane + cc * LANES) * BM
                acc = jnp.zeros((LANES,), jnp.int32)
                for e in range(E):
                    acc = acc + (off[e] <= bs).astype(jnp.int32)
                be_v[pl.ds(cc * LANES, LANES)] = acc - 1
            pltpu.sync_copy(be_v, be_hbm)
            cnt_v[...] = jnp.broadcast_to(cs[E - 1] >> BM_BITS, (LANES,))
            pltpu.sync_copy(cnt_v.at[pl.ds(0, 8)], nab_hbm)

        for j in range(TW // LANES):
            sl = pl.ds(j * LANES, LANES)
            rv1 = plsc.load_gather(off_v, [e1_v[sl]]) + r1_v[sl]
            rv2 = plsc.load_gather(off_v, [e2_v[sl]]) + r2_v[sl]
            rows1_v[sl] = rv1
            rows2_v[sl] = rv2
            rows1_m[j, :] = rv1
            rows2_m[j, :] = rv2
        rs = (pltpu.async_copy(rows1_v, rows1_hbm.at[pl.ds(tok0, TW)], sem4),
              pltpu.async_copy(rows2_v, rows2_hbm.at[pl.ds(tok0, TW)], sem4))

        pend = [None, None, None]
        for cc in range(TW // LANES):
            b = cc % 3
            rd[b].wait()
            cx1 = pltpu.async_copy(xbuf.at[b], xs_hbm.at[rows1_m.at[cc]],
                                   sems[b])
            cx2 = pltpu.async_copy(xbuf.at[b], xs_hbm.at[rows2_m.at[cc]],
                                   sems[b])
            pend[b] = (cx1, cx2)
            nc = cc + 3
            if nc < TW // LANES:
                pend[b][0].wait()
                pend[b][1].wait()
                pend[b] = None
                rd[b] = pltpu.async_copy(
                    x_hbm.at[pl.ds(tok0 + nc * LANES, LANES)], xbuf.at[b],
                    sems[b])
        for b in range(3):
            if pend[b] is not None:
                pend[b][0].wait()
                pend[b][1].wait()
        rs[0].wait()
        rs[1].wait()

    return dispatch


def _mm1_body(be_ref, nab_ref, xs_ref, w_ref, b_ref, out_ref):
    @pl.when(pl.program_id(0) < nab_ref[0])
    def _():
        acc = jnp.dot(xs_ref[...], w_ref[0],
                      preferred_element_type=jnp.float32)
        out_ref[...] = jnp.maximum(acc + b_ref[0], 0.0)


def _mm1(be, nab, xs, W1, b1):
    return pl.pallas_call(
        _mm1_body,
        grid_spec=pltpu.PrefetchScalarGridSpec(
            num_scalar_prefetch=2,
            grid=(NB,),
            in_specs=[
                pl.BlockSpec((BM, D),
                             lambda i, be, nab: (jnp.minimum(i, nab[0] - 1),
                                                 0)),
                pl.BlockSpec((1, D, H), lambda i, be, nab: (be[i], 0, 0)),
                pl.BlockSpec((1, 1, H), lambda i, be, nab: (be[i], 0, 0)),
            ],
            out_specs=pl.BlockSpec((BM, H), lambda i, be, nab: (i, 0)),
        ),
        out_shape=jax.ShapeDtypeStruct((M_PAD, H), jnp.float32),
        compiler_params=pltpu.CompilerParams(
            dimension_semantics=("arbitrary",)),
    )(be, nab, xs, W1, b1.reshape(E, 1, H))


def _mm2_body(be_ref, nab_ref, h_ref, w_ref, b_ref, out_ref):
    @pl.when(pl.program_id(0) < nab_ref[0])
    def _():
        acc = jnp.dot(h_ref[...], w_ref[0],
                      preferred_element_type=jnp.float32)
        out_ref[...] = acc + b_ref[0]


def _mm2(be, nab, h, W2, b2):
    return pl.pallas_call(
        _mm2_body,
        grid_spec=pltpu.PrefetchScalarGridSpec(
            num_scalar_prefetch=2,
            grid=(NB,),
            in_specs=[
                pl.BlockSpec((BM, H),
                             lambda i, be, nab: (jnp.minimum(i, nab[0] - 1),
                                                 0)),
                pl.BlockSpec((1, H, OUT), lambda i, be, nab: (be[i], 0, 0)),
                pl.BlockSpec((1, 1, OUT), lambda i, be, nab: (be[i], 0, 0)),
            ],
            out_specs=pl.BlockSpec((BM, OUT), lambda i, be, nab: (i, 0)),
        ),
        out_shape=jax.ShapeDtypeStruct((M_PAD, OUT), jnp.float32),
        compiler_params=pltpu.CompilerParams(
            dimension_semantics=("arbitrary",)),
    )(be, nab, h, W2, b2.reshape(E, 1, OUT))


CTOK = 8


@functools.cache
def _build_combine():
    mesh = plsc.VectorSubcoreMesh(core_axis_name="c", subcore_axis_name="s")

    @functools.partial(
        pl.kernel,
        mesh=mesh,
        out_type=jax.ShapeDtypeStruct((N, OUT), jnp.float32),
        scratch_types=[
            pltpu.VMEM((CTOK, CTOK), jnp.int32),
            pltpu.VMEM((CTOK, CTOK), jnp.int32),
            pltpu.VMEM((TW,), jnp.float32),
            pltpu.VMEM((TW,), jnp.float32),
            pltpu.VMEM((2, CTOK, OUT), jnp.float32),
            pltpu.VMEM((2, CTOK, OUT), jnp.float32),
            pltpu.VMEM((2, CTOK, OUT), jnp.float32),
            pltpu.SemaphoreType.DMA,
            pltpu.SemaphoreType.DMA,
            pltpu.SemaphoreType.DMA,
        ],
        compiler_params=pltpu.CompilerParams(needs_layout_passes=False),
    )
    def combine(y2_hbm, rows1_hbm, rows2_hbm, w1_hbm, w2_hbm, out_hbm,
                idx1_m, idx2_m, w1_v, w2_v, buf1, buf2, outb,
                sem1, sem2, sem3):
        wid = lax.axis_index("s") * NC + lax.axis_index("c")
        tok0 = wid * TW
        pltpu.sync_copy(rows1_hbm.at[pl.ds(wid * CTOK, CTOK)], idx1_m)
        pltpu.sync_copy(rows2_hbm.at[pl.ds(wid * CTOK, CTOK)], idx2_m)
        pltpu.sync_copy(w1_hbm.at[pl.ds(tok0, TW)], w1_v)
        pltpu.sync_copy(w2_hbm.at[pl.ds(tok0, TW)], w2_v)
        nchunk = TW // CTOK
        gath = [None] * nchunk
        stor = [None, None]
        gath[0] = (
            pltpu.async_copy(y2_hbm.at[idx1_m.at[0]], buf1.at[0], sem1),
            pltpu.async_copy(y2_hbm.at[idx2_m.at[0]], buf2.at[0], sem2))
        for cc in range(nchunk):
            b = cc % 2
            nb = (cc + 1) % 2
            if cc + 1 < nchunk:
                gath[cc + 1] = (
                    pltpu.async_copy(y2_hbm.at[idx1_m.at[cc + 1]],
                                     buf1.at[nb], sem1),
                    pltpu.async_copy(y2_hbm.at[idx2_m.at[cc + 1]],
                                     buf2.at[nb], sem2))
            gath[cc][0].wait()
            gath[cc][1].wait()
            if stor[b] is not None:
                stor[b].wait()
            wv1 = w1_v[pl.ds((cc // 2) * LANES, LANES)]
            wv2 = w2_v[pl.ds((cc // 2) * LANES, LANES)]
            for t in range(CTOK):
                s1 = wv1[(cc % 2) * CTOK + t]
                s2 = wv2[(cc % 2) * CTOK + t]

                def body(jj, carry, b=b, t=t, s1=s1, s2=s2):
                    sl = pl.ds(jj * LANES, LANES)
                    outb[b, t, sl] = s1 * buf1[b, t, sl] + s2 * buf2[b, t, sl]
                    return carry
                lax.fori_loop(0, OUT // LANES, body, 0)
            stor[b] = pltpu.async_copy(
                outb.at[b], out_hbm.at[pl.ds(tok0 + cc * CTOK, CTOK)], sem3)
        for b in range(2):
            if stor[b] is not None:
                stor[b].wait()

    return combine


def kernel(x, gate_W, gate_b, W1, b1, W2, b2):
    e1, e2, r1, r2, w1, w2, cnt = _routing(x, gate_W, gate_b.reshape(1, E))
    xs, rows1, rows2, be, nab = _build_dispatch()(
        e1.reshape(N), e2.reshape(N), r1.reshape(N), r2.reshape(N),
        cnt.reshape(E), x)
    h = _mm1(be, nab, xs, W1, b1)
    y2 = _mm2(be, nab, h, W2, b2)
    out = _build_combine()(
        y2, rows1.reshape(N // CTOK, CTOK), rows2.reshape(N // CTOK, CTOK),
        w1.reshape(N), w2.reshape(N))
    return out


# speedup vs baseline: 1.0279x; 1.0279x over previous
import functools

import jax
import jax.numpy as jnp
from jax import lax
from jax.experimental import pallas as pl
from jax.experimental.pallas import tpu as pltpu
from jax.experimental.pallas import tpu_sc as plsc

N = 2048
D = 2048
H = 2048
OUT = 2048
E = 8
K = 2

BM = 256
BM_BITS = 8
M_PAD = N * K + E * BM
NB = M_PAD // BM
NBP = 32

TB = 512
NTB = N // TB

NC = 2
NS = 16
NW = NC * NS
TW = N // NW
LANES = 16


def _routing_body(x_ref, gw_ref, gb_ref, e1_ref, e2_ref, r1_ref, r2_ref,
                  w1_ref, w2_ref, cnt_ref, carry_ref):
    i = pl.program_id(0)

    @pl.when(i == 0)
    def _():
        carry_ref[...] = jnp.zeros_like(carry_ref)

    s = jnp.dot(x_ref[...], gw_ref[...], preferred_element_type=jnp.float32)
    s = s + gb_ref[...]
    idx = lax.broadcasted_iota(jnp.int32, (TB, E), 1)

    m1 = jnp.max(s, axis=1, keepdims=True)
    i1 = jnp.min(jnp.where(s == m1, idx, E), axis=1, keepdims=True)
    s2 = jnp.where(idx == i1, -jnp.inf, s)
    m2 = jnp.max(s2, axis=1, keepdims=True)
    i2 = jnp.min(jnp.where(s2 == m2, idx, E), axis=1, keepdims=True)

    p = jnp.exp(s - m1)
    z = jnp.sum(p, axis=1, keepdims=True)
    e2v = jnp.exp(m2 - m1)
    denom = 1.0 + e2v + 1e-8 * z
    w1_ref[...] = 1.0 / denom
    w2_ref[...] = e2v / denom

    oh1 = (idx == i1).astype(jnp.float32)
    oh2 = (idx == i2).astype(jnp.float32)
    maskf = oh1 + oh2

    row = lax.broadcasted_iota(jnp.int32, (TB, TB), 0)
    col = lax.broadcasted_iota(jnp.int32, (TB, TB), 1)
    tri = (col < row).astype(jnp.float32)
    local = jnp.dot(tri, maskf, preferred_element_type=jnp.float32)
    rank = local + carry_ref[...]
    r1_ref[...] = jnp.sum(oh1 * rank, axis=1, keepdims=True).astype(jnp.int32)
    r2_ref[...] = jnp.sum(oh2 * rank, axis=1, keepdims=True).astype(jnp.int32)
    e1_ref[...] = i1
    e2_ref[...] = i2

    carry_new = carry_ref[...] + jnp.sum(maskf, axis=0, keepdims=True)
    carry_ref[...] = carry_new

    @pl.when(i == NTB - 1)
    def _():
        cnt_ref[...] = carry_new.astype(jnp.int32)


def _routing(x, gate_W, gate_b):
    col = pl.BlockSpec((TB, 1), lambda i: (i, 0))
    return pl.pallas_call(
        _routing_body,
        grid=(NTB,),
        in_specs=[
            pl.BlockSpec((TB, D), lambda i: (i, 0)),
            pl.BlockSpec((D, E), lambda i: (0, 0)),
            pl.BlockSpec((1, E), lambda i: (0, 0)),
        ],
        out_specs=[col, col, col, col, col, col,
                   pl.BlockSpec((1, E), lambda i: (0, 0))],
        out_shape=[
            jax.ShapeDtypeStruct((N, 1), jnp.int32),
            jax.ShapeDtypeStruct((N, 1), jnp.int32),
            jax.ShapeDtypeStruct((N, 1), jnp.int32),
            jax.ShapeDtypeStruct((N, 1), jnp.int32),
            jax.ShapeDtypeStruct((N, 1), jnp.float32),
            jax.ShapeDtypeStruct((N, 1), jnp.float32),
            jax.ShapeDtypeStruct((1, E), jnp.int32),
        ],
        scratch_shapes=[pltpu.VMEM((1, E), jnp.float32)],
        compiler_params=pltpu.CompilerParams(
            dimension_semantics=("arbitrary",)),
    )(x, gate_W, gate_b)


@functools.cache
def _build_dispatch():
    mesh = plsc.VectorSubcoreMesh(core_axis_name="c", subcore_axis_name="s")

    @functools.partial(
        pl.kernel,
        mesh=mesh,
        out_type=[
            jax.ShapeDtypeStruct((M_PAD, D), jnp.float32),
            jax.ShapeDtypeStruct((N,), jnp.int32),
            jax.ShapeDtypeStruct((N,), jnp.int32),
            jax.ShapeDtypeStruct((NBP,), jnp.int32),
            jax.ShapeDtypeStruct((8,), jnp.int32),
        ],
        scratch_types=[
            pltpu.VMEM((LANES,), jnp.int32),
            pltpu.VMEM((LANES,), jnp.int32),
            pltpu.VMEM((NBP,), jnp.int32),
            pltpu.VMEM((TW,), jnp.int32),
            pltpu.VMEM((TW,), jnp.int32),
            pltpu.VMEM((TW,), jnp.int32),
            pltpu.VMEM((TW,), jnp.int32),
            pltpu.VMEM((TW,), jnp.int32),
            pltpu.VMEM((TW,), jnp.int32),
            pltpu.VMEM((TW // LANES, LANES), jnp.int32),
            pltpu.VMEM((TW // LANES, LANES), jnp.int32),
            pltpu.VMEM((3, LANES, D), jnp.float32),
            pltpu.SemaphoreType.DMA,
            pltpu.SemaphoreType.DMA,
            pltpu.SemaphoreType.DMA,
            pltpu.SemaphoreType.DMA,
        ],
        compiler_params=pltpu.CompilerParams(needs_layout_passes=False),
    )
    def dispatch(e1_hbm, e2_hbm, r1_hbm, r2_hbm, cnt_hbm,
                 x_hbm, xs_hbm, rows1_hbm, rows2_hbm, be_hbm, nab_hbm,
                 cnt_v, off_v, be_v, e1_v, e2_v, r1_v, r2_v,
                 rows1_v, rows2_v, rows1_m, rows2_m, xbuf,
                 sem1, sem2, sem3, sem4):
        wid = lax.axis_index("s") * NC + lax.axis_index("c")
        tok0 = wid * TW

        sems = [sem1, sem2, sem3]
        rd = [pltpu.async_copy(
            x_hbm.at[pl.ds(tok0 + b * LANES, LANES)], xbuf.at[b], sems[b])
            for b in range(3)]

        hc = (pltpu.async_copy(cnt_hbm, cnt_v.at[pl.ds(0, E)], sem4),
              pltpu.async_copy(e1_hbm.at[pl.ds(tok0, TW)], e1_v, sem4),
              pltpu.async_copy(e2_hbm.at[pl.ds(tok0, TW)], e2_v, sem4),
              pltpu.async_copy(r1_hbm.at[pl.ds(tok0, TW)], r1_v, sem4),
              pltpu.async_copy(r2_hbm.at[pl.ds(tok0, TW)], r2_v, sem4))
        for h in hc:
            h.wait()
        lane = lax.iota(jnp.int32, LANES)
        c = cnt_v[...]
        cpad = jnp.where(lane < E, ((c + (BM - 1)) >> BM_BITS) << BM_BITS, 0)
        cs = plsc.cumsum(cpad)
        off = cs - cpad
        off_v[...] = off

        @pl.when(wid == 0)
        def _():
            for cc in range(NBP // LANES):
                bs = (lane + cc * LANES) * BM
                acc = jnp.zeros((LANES,), jnp.int32)
                for e in range(E):
                    acc = acc + (off[e] <= bs).astype(jnp.int32)
                be_v[pl.ds(cc * LANES, LANES)] = acc - 1
            pltpu.sync_copy(be_v, be_hbm)
            cnt_v[...] = jnp.broadcast_to(cs[E - 1] >> BM_BITS, (LANES,))
            pltpu.sync_copy(cnt_v.at[pl.ds(0, 8)], nab_hbm)

        for j in range(TW // LANES):
            sl = pl.ds(j * LANES, LANES)
            rv1 = plsc.load_gather(off_v, [e1_v[sl]]) + r1_v[sl]
            rv2 = plsc.load_gather(off_v, [e2_v[sl]]) + r2_v[sl]
            rows1_v[sl] = rv1
            rows2_v[sl] = rv2
            rows1_m[j, :] = rv1
            rows2_m[j, :] = rv2
        rs = (pltpu.async_copy(rows1_v, rows1_hbm.at[pl.ds(tok0, TW)], sem4),
              pltpu.async_copy(rows2_v, rows2_hbm.at[pl.ds(tok0, TW)], sem4))

        pend = [None, None, None]
        for cc in range(TW // LANES):
            b = cc % 3
            rd[b].wait()
            cx1 = pltpu.async_copy(xbuf.at[b], xs_hbm.at[rows1_m.at[cc]],
                                   sems[b])
            cx2 = pltpu.async_copy(xbuf.at[b], xs_hbm.at[rows2_m.at[cc]],
                                   sems[b])
            pend[b] = (cx1, cx2)
            nc = cc + 3
            if nc < TW // LANES:
                pend[b][0].wait()
                pend[b][1].wait()
                pend[b] = None
                rd[b] = pltpu.async_copy(
                    x_hbm.at[pl.ds(tok0 + nc * LANES, LANES)], xbuf.at[b],
                    sems[b])
        for b in range(3):
            if pend[b] is not None:
                pend[b][0].wait()
                pend[b][1].wait()
        rs[0].wait()
        rs[1].wait()

    return dispatch


def _mm1_body(be_ref, nab_ref, xs_ref, w_ref, b_ref, out_ref):
    @pl.when(pl.program_id(0) < nab_ref[0])
    def _():
        acc = jnp.dot(xs_ref[...], w_ref[0],
                      preferred_element_type=jnp.float32)
        out_ref[...] = jnp.maximum(acc + b_ref[0], 0.0)


def _mm1(be, nab, xs, W1, b1):
    return pl.pallas_call(
        _mm1_body,
        grid_spec=pltpu.PrefetchScalarGridSpec(
            num_scalar_prefetch=2,
            grid=(NB,),
            in_specs=[
                pl.BlockSpec((BM, D),
                             lambda i, be, nab: (jnp.minimum(i, nab[0] - 1),
                                                 0)),
                pl.BlockSpec((1, D, H), lambda i, be, nab: (be[i], 0, 0)),
                pl.BlockSpec((1, 1, H), lambda i, be, nab: (be[i], 0, 0)),
            ],
            out_specs=pl.BlockSpec((BM, H), lambda i, be, nab: (i, 0)),
        ),
        out_shape=jax.ShapeDtypeStruct((M_PAD, H), jnp.float32),
        compiler_params=pltpu.CompilerParams(
            dimension_semantics=("arbitrary",)),
    )(be, nab, xs, W1, b1.reshape(E, 1, H))


def _mm2_body(be_ref, nab_ref, h_ref, w_ref, b_ref, out_ref):
    @pl.when(pl.program_id(0) < nab_ref[0])
    def _():
        acc = jnp.dot(h_ref[...], w_ref[0],
                      preferred_element_type=jnp.float32)
        out_ref[...] = acc + b_ref[0]


def _mm2(be, nab, h, W2, b2):
    return pl.pallas_call(
        _mm2_body,
        grid_spec=pltpu.PrefetchScalarGridSpec(
            num_scalar_prefetch=2,
            grid=(NB,),
            in_specs=[
                pl.BlockSpec((BM, H),
                             lambda i, be, nab: (jnp.minimum(i, nab[0] - 1),
                                                 0)),
                pl.BlockSpec((1, H, OUT), lambda i, be, nab: (be[i], 0, 0)),
                pl.BlockSpec((1, 1, OUT), lambda i, be, nab: (be[i], 0, 0)),
            ],
            out_specs=pl.BlockSpec((BM, OUT), lambda i, be, nab: (i, 0)),
        ),
        out_shape=jax.ShapeDtypeStruct((M_PAD, OUT), jnp.float32),
        compiler_params=pltpu.CompilerParams(
            dimension_semantics=("arbitrary",)),
    )(be, nab, h, W2, b2.reshape(E, 1, OUT))


CTOK = 8


@functools.cache
def _build_combine():
    mesh = plsc.VectorSubcoreMesh(core_axis_name="c", subcore_axis_name="s")

    @functools.partial(
        pl.kernel,
        mesh=mesh,
        out_type=jax.ShapeDtypeStruct((N, OUT), jnp.float32),
        scratch_types=[
            pltpu.VMEM((CTOK, CTOK), jnp.int32),
            pltpu.VMEM((CTOK, CTOK), jnp.int32),
            pltpu.VMEM((TW,), jnp.float32),
            pltpu.VMEM((TW,), jnp.float32),
            pltpu.VMEM((2, CTOK, OUT), jnp.float32),
            pltpu.VMEM((2, CTOK, OUT), jnp.float32),
            pltpu.VMEM((2, CTOK, OUT), jnp.float32),
            pltpu.SemaphoreType.DMA,
            pltpu.SemaphoreType.DMA,
            pltpu.SemaphoreType.DMA,
        ],
        compiler_params=pltpu.CompilerParams(needs_layout_passes=False),
    )
    def combine(y2_hbm, rows1_hbm, rows2_hbm, w1_hbm, w2_hbm, out_hbm,
                idx1_m, idx2_m, w1_v, w2_v, buf1, buf2, outb,
                sem1, sem2, sem3):
        wid = lax.axis_index("s") * NC + lax.axis_index("c")
        tok0 = wid * TW
        pltpu.sync_copy(rows1_hbm.at[pl.ds(wid * CTOK, CTOK)], idx1_m)
        pltpu.sync_copy(rows2_hbm.at[pl.ds(wid * CTOK, CTOK)], idx2_m)
        pltpu.sync_copy(w1_hbm.at[pl.ds(tok0, TW)], w1_v)
        pltpu.sync_copy(w2_hbm.at[pl.ds(tok0, TW)], w2_v)
        nchunk = TW // CTOK
        gath = [None] * nchunk
        stor = [None, None]
        gath[0] = (
            pltpu.async_copy(y2_hbm.at[idx1_m.at[0]], buf1.at[0], sem1),
            pltpu.async_copy(y2_hbm.at[idx2_m.at[0]], buf2.at[0], sem2))
        for cc in range(nchunk):
            b = cc % 2
            nb = (cc + 1) % 2
            if cc + 1 < nchunk:
                gath[cc + 1] = (
                    pltpu.async_copy(y2_hbm.at[idx1_m.at[cc + 1]],
                                     buf1.at[nb], sem1),
                    pltpu.async_copy(y2_hbm.at[idx2_m.at[cc + 1]],
                                     buf2.at[nb], sem2))
            gath[cc][0].wait()
            gath[cc][1].wait()
            if stor[b] is not None:
                stor[b].wait()
            wv1 = w1_v[pl.ds((cc // 2) * LANES, LANES)]
            wv2 = w2_v[pl.ds((cc // 2) * LANES, LANES)]
            for t in range(CTOK):
                s1 = wv1[(cc % 2) * CTOK + t]
                s2 = wv2[(cc % 2) * CTOK + t]

                def body(jj, carry, b=b, t=t, s1=s1, s2=s2):
                    sl = pl.ds(jj * LANES, LANES)
                    outb[b, t, sl] = s1 * buf1[b, t, sl] + s2 * buf2[b, t, sl]
                    return carry
                lax.fori_loop(0, OUT // LANES, body, 0)
            stor[b] = pltpu.async_copy(
                outb.at[b], out_hbm.at[pl.ds(tok0 + cc * CTOK, CTOK)], sem3)
        for b in range(2):
            if stor[b] is not None:
                stor[b].wait()

    return combine


def kernel(x, gate_W, gate_b, W1, b1, W2, b2):
    e1, e2, r1, r2, w1, w2, cnt = _routing(x, gate_W, gate_b.reshape(1, E))
    xs, rows1, rows2, be, nab = _build_dispatch()(
        e1.reshape(N), e2.reshape(N), r1.reshape(N), r2.reshape(N),
        cnt.reshape(E), x)
    h = _mm1(be, nab, xs, W1, b1)
    y2 = _mm2(be, nab, h, W2, b2)
    out = _build_combine()(
        y2, rows1.reshape(N // CTOK, CTOK), rows2.reshape(N // CTOK, CTOK),
        w1.reshape(N), w2.reshape(N))
    return out
